# baseline (speedup 1.0000x reference)
import functools

import jax
import jax.numpy as jnp
from jax import lax
from jax.experimental import pallas as pl
from jax.experimental.pallas import tpu as pltpu
from jax.experimental.pallas import tpu_sc as plsc

NC = 2
NS = 16
NW = NC * NS
LANES = 16

N = 10000
F_IN = 128
H = 64
C_OUT = 8
E = 320000

NPAD = 10240
RPT = NPAD // NS
BLK = 128
EPT = 10240
NBLK = EPT // BLK
EPAD = EPT * NW
NBUF = 4
ZR = 128
BR = 512


def _sc_mesh():
    return plsc.VectorSubcoreMesh(core_axis_name="c", subcore_axis_name="s")


def _degree_body(dst_hbm, out_hbm, dst_v, deg_l, acc16, res, sh):
    c = lax.axis_index("c")
    s = lax.axis_index("s")
    w = c * NS + s
    pltpu.sync_copy(dst_hbm.at[w], dst_v)

    zeros = jnp.zeros((LANES,), jnp.float32)

    def zero(k, _):
        deg_l[pl.ds(k * LANES, LANES)] = zeros
        return 0

    lax.fori_loop(0, NPAD // LANES, zero, 0)

    ones = jnp.ones((LANES,), jnp.float32)

    def acc(k, _):
        idx = dst_v[pl.ds(k * LANES, LANES)]
        plsc.addupdate_scatter(deg_l, [idx], ones)
        return 0

    lax.fori_loop(0, EPT // LANES, acc, 0)

    pltpu.sync_copy(deg_l, sh.at[s])
    plsc.subcore_barrier()

    cpt = NPAD // NS
    for r in range(NS):
        pltpu.sync_copy(sh.at[r, pl.ds(s * cpt, cpt)], acc16.at[r])

    def red(k, _):
        a = acc16[0, pl.ds(k * LANES, LANES)]
        for r in range(1, NS):
            a = a + acc16[r, pl.ds(k * LANES, LANES)]
        res[pl.ds(k * LANES, LANES)] = a
        return 0

    lax.fori_loop(0, cpt // LANES, red, 0)
    pltpu.sync_copy(res, out_hbm.at[c, pl.ds(s * (NPAD // NS), NPAD // NS)])


def _sc_degree(dst_tiles):
    k = functools.partial(
        pl.kernel,
        out_type=jax.ShapeDtypeStruct((NC, NPAD), jnp.float32),
        mesh=_sc_mesh(),
        compiler_params=pltpu.CompilerParams(needs_layout_passes=False),
        scratch_types=[
            pltpu.VMEM((EPT,), jnp.int32),
            pltpu.VMEM((NPAD,), jnp.float32),
            pltpu.VMEM((NS, NPAD // NS), jnp.float32),
            pltpu.VMEM((NPAD // NS,), jnp.float32),
            pltpu.VMEM_SHARED((NS, NPAD), jnp.float32),
        ],
    )(_degree_body)
    return k(dst_tiles)


def _agg_body(g_hbm, src_hbm, dst_hbm, out_hbm, src_v, dst_v,
              b0, b1, b2, b3, zbuf, agg_sh, s0, s1, s2, s3):
    bufs = (b0, b1, b2, b3)
    sems = (s0, s1, s2, s3)
    c = lax.axis_index("c")
    s = lax.axis_index("s")
    w = c * NS + s
    pltpu.sync_copy(src_hbm.at[w], src_v)
    pltpu.sync_copy(dst_hbm.at[w], dst_v)

    zeros = jnp.zeros((LANES,), jnp.float32)

    def zero(r, _):
        for l in range(H // LANES):
            zbuf[r, pl.ds(l * LANES, LANES)] = zeros
        return 0

    lax.fori_loop(0, ZR, zero, 0)
    for q in range(RPT // ZR):
        pltpu.sync_copy(zbuf, agg_sh.at[pl.ds(s * RPT + q * ZR, ZR)])
    plsc.subcore_barrier()

    for b in range(NBUF):
        pltpu.async_copy(g_hbm.at[src_v.at[b]], bufs[b], sems[b])

    def step(j4, _):
        for b in range(NBUF):
            j = NBUF * j4 + b
            pltpu.make_async_copy(g_hbm.at[src_v.at[j]], bufs[b],
                                  sems[b]).wait()
            pltpu.sync_copy(bufs[b], agg_sh.at[dst_v.at[j]], add=True)
            pltpu.async_copy(g_hbm.at[src_v.at[j + NBUF]], bufs[b], sems[b])
        return 0

    lax.fori_loop(0, NBLK // NBUF, step, 0)
    for b in range(NBUF):
        pltpu.make_async_copy(g_hbm.at[src_v.at[NBLK + b]], bufs[b],
                              sems[b]).wait()
    plsc.subcore_barrier()
    pltpu.sync_copy(agg_sh.at[pl.ds(s * RPT, RPT)],
                    out_hbm.at[c, pl.ds(s * RPT, RPT)])


def _sc_aggregate(g, src_tiles, dst_tiles):
    k = functools.partial(
        pl.kernel,
        out_type=jax.ShapeDtypeStruct((NC, NPAD, H), jnp.float32),
        mesh=_sc_mesh(),
        compiler_params=pltpu.CompilerParams(use_tc_tiling_on_sc=False),
        scratch_types=[
            pltpu.VMEM((NBLK + NBUF, BLK), jnp.int32),
            pltpu.VMEM((NBLK + NBUF, BLK), jnp.int32),
            pltpu.VMEM((BLK, H), jnp.float32),
            pltpu.VMEM((BLK, H), jnp.float32),
            pltpu.VMEM((BLK, H), jnp.float32),
            pltpu.VMEM((BLK, H), jnp.float32),
            pltpu.VMEM((ZR, H), jnp.float32),
            pltpu.VMEM_SHARED((NPAD, H), jnp.float32),
            pltpu.SemaphoreType.DMA,
            pltpu.SemaphoreType.DMA,
            pltpu.SemaphoreType.DMA,
            pltpu.SemaphoreType.DMA,
        ],
    )(_agg_body)
    return k(g, src_tiles, dst_tiles)


def _tc1_body(x_ref, w_ref, d_ref, g_ref, dis_ref):
    p = jnp.dot(x_ref[...], w_ref[...], preferred_element_type=jnp.float32)
    deg = d_ref[0, :] + d_ref[1, :] + 1.0
    dis = lax.rsqrt(deg)
    g_ref[...] = p * dis[:, None]
    dis_ref[...] = dis


def _tc1(xp, W1, degp):
    return pl.pallas_call(
        _tc1_body,
        grid=(NPAD // BR,),
        in_specs=[
            pl.BlockSpec((BR, F_IN), lambda i: (i, 0)),
            pl.BlockSpec((F_IN, H), lambda i: (0, 0)),
            pl.BlockSpec((NC, BR), lambda i: (0, i)),
        ],
        out_specs=[
            pl.BlockSpec((BR, H), lambda i: (i, 0)),
            pl.BlockSpec((BR,), lambda i: (i,)),
        ],
        out_shape=[
            jax.ShapeDtypeStruct((NPAD, H), jnp.float32),
            jax.ShapeDtypeStruct((NPAD,), jnp.float32),
        ],
    )(xp, W1, degp)


def _tc2_body(a_ref, g_ref, dis_ref, w_ref, b_ref, o_ref):
    dis = dis_ref[...]
    h = a_ref[0] + a_ref[1] + g_ref[...]
    h = jnp.maximum(h * dis[:, None] + b_ref[...][None, :], 0.0)
    p = jnp.dot(h, w_ref[...], preferred_element_type=jnp.float32)
    o_ref[...] = p * dis[:, None]


def _tc2(agg, g1, dis, W2, b1):
    return pl.pallas_call(
        _tc2_body,
        grid=(NPAD // BR,),
        in_specs=[
            pl.BlockSpec((NC, BR, H), lambda i: (0, i, 0)),
            pl.BlockSpec((BR, H), lambda i: (i, 0)),
            pl.BlockSpec((BR,), lambda i: (i,)),
            pl.BlockSpec((H, H), lambda i: (0, 0)),
            pl.BlockSpec((H,), lambda i: (0,)),
        ],
        out_specs=pl.BlockSpec((BR, H), lambda i: (i, 0)),
        out_shape=jax.ShapeDtypeStruct((NPAD, H), jnp.float32),
    )(agg, g1, dis, W2, b1)


def _tc3_body(a_ref, g_ref, dis_ref, w_ref, b2_ref, bc_ref, o_ref):
    dis = dis_ref[...]
    h = a_ref[0] + a_ref[1] + g_ref[...]
    h = jnp.maximum(h * dis[:, None] + b2_ref[...][None, :], 0.0)
    t = jnp.dot(h, w_ref[...], preferred_element_type=jnp.float32)
    t = t + bc_ref[...][None, :]
    m = jnp.max(t, axis=1, keepdims=True)
    lse = m + jnp.log(jnp.sum(jnp.exp(t - m), axis=1, keepdims=True))
    o_ref[...] = t - lse


def _tc3(agg, g2, dis, Wc, b2, bc):
    return pl.pallas_call(
        _tc3_body,
        grid=(NPAD // BR,),
        in_specs=[
            pl.BlockSpec((NC, BR, H), lambda i: (0, i, 0)),
            pl.BlockSpec((BR, H), lambda i: (i, 0)),
            pl.BlockSpec((BR,), lambda i: (i,)),
            pl.BlockSpec((H, C_OUT), lambda i: (0, 0)),
            pl.BlockSpec((H,), lambda i: (0,)),
            pl.BlockSpec((C_OUT,), lambda i: (0,)),
        ],
        out_specs=pl.BlockSpec((BR, C_OUT), lambda i: (i, 0)),
        out_shape=jax.ShapeDtypeStruct((NPAD, C_OUT), jnp.float32),
    )(agg, g2, dis, Wc, b2, bc)


def kernel(x, edge_index, W1, b1, W2, b2, Wc, bc):
    pad = jnp.full((EPAD - E,), N, dtype=jnp.int32)
    src_flat = jnp.concatenate([edge_index[0], pad])
    dst_flat = jnp.concatenate([edge_index[1], pad])
    dummy = jnp.full((NW, NBUF, BLK), N, dtype=jnp.int32)
    src_tiles = jnp.concatenate(
        [src_flat.reshape(NW, NBLK, BLK), dummy], axis=1)
    dst_tiles = jnp.concatenate(
        [dst_flat.reshape(NW, NBLK, BLK), dummy], axis=1)
    dst_tiles_flat = dst_flat.reshape(NW, EPT)

    xp = jnp.pad(x, ((0, NPAD - N), (0, 0)))

    degp = _sc_degree(dst_tiles_flat)
    g1, dis = _tc1(xp, W1, degp)
    agg1 = _sc_aggregate(g1, src_tiles, dst_tiles)
    g2 = _tc2(agg1, g1, dis, W2, b1)
    agg2 = _sc_aggregate(g2, src_tiles, dst_tiles)
    logp = _tc3(agg2, g2, dis, Wc, b2, bc)
    return logp[:N]


# speedup vs baseline: 3.4862x; 3.4862x over previous
import functools

import jax
import jax.numpy as jnp
from jax import lax
from jax.experimental import pallas as pl
from jax.experimental.pallas import tpu as pltpu
from jax.experimental.pallas import tpu_sc as plsc

NC = 2
NS = 16
NW = NC * NS
LANES = 16

N = 10000
F_IN = 128
H = 64
C_OUT = 8
E = 320000

NPAD = 10240
RPT = NPAD // NS
BLK = 128
EPT = 10240
NBLK = EPT // BLK
EPAD = EPT * NW
NBUF = 4
ZR = 128
BR = 512


def _sc_mesh():
    return plsc.VectorSubcoreMesh(core_axis_name="c", subcore_axis_name="s")


def _degree_body(dst_hbm, out_hbm, dst_v, deg_l, acc16, res, sh):
    c = lax.axis_index("c")
    s = lax.axis_index("s")
    w = c * NS + s
    pltpu.sync_copy(dst_hbm.at[w], dst_v)

    zeros = jnp.zeros((LANES,), jnp.float32)

    def zero(k, _):
        deg_l[pl.ds(k * LANES, LANES)] = zeros
        return 0

    lax.fori_loop(0, NPAD // LANES, zero, 0)

    ones = jnp.ones((LANES,), jnp.float32)

    def acc(k, _):
        idx = dst_v[pl.ds(k * LANES, LANES)]
        plsc.addupdate_scatter(deg_l, [idx], ones)
        return 0

    lax.fori_loop(0, EPT // LANES, acc, 0)

    pltpu.sync_copy(deg_l, sh.at[s])
    plsc.subcore_barrier()

    cpt = NPAD // NS
    for r in range(NS):
        pltpu.sync_copy(sh.at[r, pl.ds(s * cpt, cpt)], acc16.at[r])

    def red(k, _):
        a = acc16[0, pl.ds(k * LANES, LANES)]
        for r in range(1, NS):
            a = a + acc16[r, pl.ds(k * LANES, LANES)]
        res[pl.ds(k * LANES, LANES)] = a
        return 0

    lax.fori_loop(0, cpt // LANES, red, 0)
    pltpu.sync_copy(res, out_hbm.at[c, pl.ds(s * (NPAD // NS), NPAD // NS)])


def _sc_degree(dst_tiles):
    k = functools.partial(
        pl.kernel,
        out_type=jax.ShapeDtypeStruct((NC, NPAD), jnp.float32),
        mesh=_sc_mesh(),
        compiler_params=pltpu.CompilerParams(needs_layout_passes=False),
        scratch_types=[
            pltpu.VMEM((EPT,), jnp.int32),
            pltpu.VMEM((NPAD,), jnp.float32),
            pltpu.VMEM((NS, NPAD // NS), jnp.float32),
            pltpu.VMEM((NPAD // NS,), jnp.float32),
            pltpu.VMEM_SHARED((NS, NPAD), jnp.float32),
        ],
    )(_degree_body)
    return k(dst_tiles)


def _agg_body(g_hbm, src_hbm, dst_hbm, out_hbm, src_v, dst_v,
              b0, b1, b2, b3, zbuf, g_sh, agg_sh, s0, s1, s2, s3):
    bufs = (b0, b1, b2, b3)
    sems = (s0, s1, s2, s3)
    c = lax.axis_index("c")
    s = lax.axis_index("s")
    w = c * NS + s
    pltpu.sync_copy(src_hbm.at[w], src_v)
    pltpu.sync_copy(dst_hbm.at[w], dst_v)

    zeros = jnp.zeros((LANES,), jnp.float32)

    def zero(r, _):
        for l in range(H // LANES):
            zbuf[r, pl.ds(l * LANES, LANES)] = zeros
        return 0

    lax.fori_loop(0, ZR, zero, 0)
    for q in range(RPT // ZR):
        pltpu.sync_copy(zbuf, agg_sh.at[pl.ds(s * RPT + q * ZR, ZR)])
    plsc.subcore_barrier()

    pltpu.sync_copy(g_hbm.at[pl.ds(s * RPT, RPT)],
                    g_sh.at[pl.ds(s * RPT, RPT)])
    plsc.subcore_barrier()

    def step(j, _):
        cp = pltpu.async_copy(g_sh.at[src_v.at[j]], bufs[0], sems[0])
        cp.wait()
        pltpu.sync_copy(bufs[0], agg_sh.at[dst_v.at[j]], add=True)
        return 0

    lax.fori_loop(0, NBLK, step, 0)
    plsc.subcore_barrier()
    pltpu.sync_copy(agg_sh.at[pl.ds(s * RPT, RPT)],
                    out_hbm.at[c, pl.ds(s * RPT, RPT)])


def _sc_aggregate(g, src_tiles, dst_tiles):
    k = functools.partial(
        pl.kernel,
        out_type=jax.ShapeDtypeStruct((NC, NPAD, H), jnp.float32),
        mesh=_sc_mesh(),
        compiler_params=pltpu.CompilerParams(use_tc_tiling_on_sc=False),
        scratch_types=[
            pltpu.VMEM((NBLK + NBUF, BLK), jnp.int32),
            pltpu.VMEM((NBLK + NBUF, BLK), jnp.int32),
            pltpu.VMEM((BLK, H), jnp.float32),
            pltpu.VMEM((BLK, H), jnp.float32),
            pltpu.VMEM((BLK, H), jnp.float32),
            pltpu.VMEM((BLK, H), jnp.float32),
            pltpu.VMEM((ZR, H), jnp.float32),
            pltpu.VMEM_SHARED((NPAD, H), jnp.float32),
            pltpu.VMEM_SHARED((NPAD, H), jnp.float32),
            pltpu.SemaphoreType.DMA,
            pltpu.SemaphoreType.DMA,
            pltpu.SemaphoreType.DMA,
            pltpu.SemaphoreType.DMA,
        ],
    )(_agg_body)
    return k(g, src_tiles, dst_tiles)


def _tc1_body(x_ref, w_ref, d_ref, g_ref, dis_ref):
    p = jnp.dot(x_ref[...], w_ref[...], preferred_element_type=jnp.float32)
    deg = d_ref[0, :] + d_ref[1, :] + 1.0
    dis = lax.rsqrt(deg)
    g_ref[...] = p * dis[:, None]
    dis_ref[...] = dis


def _tc1(xp, W1, degp):
    return pl.pallas_call(
        _tc1_body,
        grid=(NPAD // BR,),
        in_specs=[
            pl.BlockSpec((BR, F_IN), lambda i: (i, 0)),
            pl.BlockSpec((F_IN, H), lambda i: (0, 0)),
            pl.BlockSpec((NC, BR), lambda i: (0, i)),
        ],
        out_specs=[
            pl.BlockSpec((BR, H), lambda i: (i, 0)),
            pl.BlockSpec((BR,), lambda i: (i,)),
        ],
        out_shape=[
            jax.ShapeDtypeStruct((NPAD, H), jnp.float32),
            jax.ShapeDtypeStruct((NPAD,), jnp.float32),
        ],
    )(xp, W1, degp)


def _tc2_body(a_ref, g_ref, dis_ref, w_ref, b_ref, o_ref):
    dis = dis_ref[...]
    h = a_ref[0] + a_ref[1] + g_ref[...]
    h = jnp.maximum(h * dis[:, None] + b_ref[...][None, :], 0.0)
    p = jnp.dot(h, w_ref[...], preferred_element_type=jnp.float32)
    o_ref[...] = p * dis[:, None]


def _tc2(agg, g1, dis, W2, b1):
    return pl.pallas_call(
        _tc2_body,
        grid=(NPAD // BR,),
        in_specs=[
            pl.BlockSpec((NC, BR, H), lambda i: (0, i, 0)),
            pl.BlockSpec((BR, H), lambda i: (i, 0)),
            pl.BlockSpec((BR,), lambda i: (i,)),
            pl.BlockSpec((H, H), lambda i: (0, 0)),
            pl.BlockSpec((H,), lambda i: (0,)),
        ],
        out_specs=pl.BlockSpec((BR, H), lambda i: (i, 0)),
        out_shape=jax.ShapeDtypeStruct((NPAD, H), jnp.float32),
    )(agg, g1, dis, W2, b1)


def _tc3_body(a_ref, g_ref, dis_ref, w_ref, b2_ref, bc_ref, o_ref):
    dis = dis_ref[...]
    h = a_ref[0] + a_ref[1] + g_ref[...]
    h = jnp.maximum(h * dis[:, None] + b2_ref[...][None, :], 0.0)
    t = jnp.dot(h, w_ref[...], preferred_element_type=jnp.float32)
    t = t + bc_ref[...][None, :]
    m = jnp.max(t, axis=1, keepdims=True)
    lse = m + jnp.log(jnp.sum(jnp.exp(t - m), axis=1, keepdims=True))
    o_ref[...] = t - lse


def _tc3(agg, g2, dis, Wc, b2, bc):
    return pl.pallas_call(
        _tc3_body,
        grid=(NPAD // BR,),
        in_specs=[
            pl.BlockSpec((NC, BR, H), lambda i: (0, i, 0)),
            pl.BlockSpec((BR, H), lambda i: (i, 0)),
            pl.BlockSpec((BR,), lambda i: (i,)),
            pl.BlockSpec((H, C_OUT), lambda i: (0, 0)),
            pl.BlockSpec((H,), lambda i: (0,)),
            pl.BlockSpec((C_OUT,), lambda i: (0,)),
        ],
        out_specs=pl.BlockSpec((BR, C_OUT), lambda i: (i, 0)),
        out_shape=jax.ShapeDtypeStruct((NPAD, C_OUT), jnp.float32),
    )(agg, g2, dis, Wc, b2, bc)


def kernel(x, edge_index, W1, b1, W2, b2, Wc, bc):
    pad = jnp.full((EPAD - E,), N, dtype=jnp.int32)
    src_flat = jnp.concatenate([edge_index[0], pad])
    dst_flat = jnp.concatenate([edge_index[1], pad])
    dummy = jnp.full((NW, NBUF, BLK), N, dtype=jnp.int32)
    src_tiles = jnp.concatenate(
        [src_flat.reshape(NW, NBLK, BLK), dummy], axis=1)
    dst_tiles = jnp.concatenate(
        [dst_flat.reshape(NW, NBLK, BLK), dummy], axis=1)
    dst_tiles_flat = dst_flat.reshape(NW, EPT)

    xp = jnp.pad(x, ((0, NPAD - N), (0, 0)))

    degp = _sc_degree(dst_tiles_flat)
    g1, dis = _tc1(xp, W1, degp)
    agg1 = _sc_aggregate(g1, src_tiles, dst_tiles)
    g2 = _tc2(agg1, g1, dis, W2, b1)
    agg2 = _sc_aggregate(g2, src_tiles, dst_tiles)
    logp = _tc3(agg2, g2, dis, Wc, b2, bc)
    return logp[:N]


# speedup vs baseline: 3.5328x; 1.0133x over previous
import functools

import jax
import jax.numpy as jnp
from jax import lax
from jax.experimental import pallas as pl
from jax.experimental.pallas import tpu as pltpu
from jax.experimental.pallas import tpu_sc as plsc

NC = 2
NS = 16
NW = NC * NS
LANES = 16

N = 10000
F_IN = 128
H = 64
C_OUT = 8
E = 320000

NPAD = 10240
RPT = NPAD // NS
BLK = 128
EPT = 10240
NBLK = EPT // BLK
EPAD = EPT * NW
NBUF = 4
ZR = 128
BR = 512


def _sc_mesh():
    return plsc.VectorSubcoreMesh(core_axis_name="c", subcore_axis_name="s")


def _degree_body(dst_hbm, out_hbm, dst_v, deg_l, acc16, res, sh):
    c = lax.axis_index("c")
    s = lax.axis_index("s")
    w = c * NS + s
    pltpu.sync_copy(dst_hbm.at[w], dst_v)

    zeros = jnp.zeros((LANES,), jnp.float32)

    def zero(k, _):
        deg_l[pl.ds(k * LANES, LANES)] = zeros
        return 0

    lax.fori_loop(0, NPAD // LANES, zero, 0)

    ones = jnp.ones((LANES,), jnp.float32)

    def acc(k, _):
        idx = dst_v[pl.ds(k * LANES, LANES)]
        plsc.addupdate_scatter(deg_l, [idx], ones)
        return 0

    lax.fori_loop(0, EPT // LANES, acc, 0)

    pltpu.sync_copy(deg_l, sh.at[s])
    plsc.subcore_barrier()

    cpt = NPAD // NS
    for r in range(NS):
        pltpu.sync_copy(sh.at[r, pl.ds(s * cpt, cpt)], acc16.at[r])

    def red(k, _):
        a = acc16[0, pl.ds(k * LANES, LANES)]
        for r in range(1, NS):
            a = a + acc16[r, pl.ds(k * LANES, LANES)]
        res[pl.ds(k * LANES, LANES)] = a
        return 0

    lax.fori_loop(0, cpt // LANES, red, 0)
    pltpu.sync_copy(res, out_hbm.at[c, pl.ds(s * (NPAD // NS), NPAD // NS)])


def _sc_degree(dst_tiles):
    k = functools.partial(
        pl.kernel,
        out_type=jax.ShapeDtypeStruct((NC, NPAD), jnp.float32),
        mesh=_sc_mesh(),
        compiler_params=pltpu.CompilerParams(needs_layout_passes=False),
        scratch_types=[
            pltpu.VMEM((EPT,), jnp.int32),
            pltpu.VMEM((NPAD,), jnp.float32),
            pltpu.VMEM((NS, NPAD // NS), jnp.float32),
            pltpu.VMEM((NPAD // NS,), jnp.float32),
            pltpu.VMEM_SHARED((NS, NPAD), jnp.float32),
        ],
    )(_degree_body)
    return k(dst_tiles)


def _agg_body(g_hbm, src_hbm, dst_hbm, out_hbm, src_v, dst_v,
              b0, b1, b2, b3, zbuf, g_sh, agg_sh, s0, s1, s2, s3):
    bufs = (b0, b1, b2, b3)
    sems = (s0, s1, s2, s3)
    c = lax.axis_index("c")
    s = lax.axis_index("s")
    w = c * NS + s
    pltpu.sync_copy(src_hbm.at[w], src_v)
    pltpu.sync_copy(dst_hbm.at[w], dst_v)

    zeros = jnp.zeros((LANES,), jnp.float32)

    def zero(r, _):
        for l in range(H // LANES):
            zbuf[r, pl.ds(l * LANES, LANES)] = zeros
        return 0

    lax.fori_loop(0, ZR, zero, 0)
    for q in range(RPT // ZR):
        pltpu.sync_copy(zbuf, agg_sh.at[pl.ds(s * RPT + q * ZR, ZR)])
    plsc.subcore_barrier()

    pltpu.sync_copy(g_hbm.at[pl.ds(s * RPT, RPT)],
                    g_sh.at[pl.ds(s * RPT, RPT)])
    plsc.subcore_barrier()

    def step(j2, _):
        j = 2 * j2
        cp0 = pltpu.async_copy(g_sh.at[src_v.at[j]], bufs[0], sems[0])
        cp1 = pltpu.async_copy(g_sh.at[src_v.at[j + 1]], bufs[1], sems[1])
        cp0.wait()
        pltpu.sync_copy(bufs[0], agg_sh.at[dst_v.at[j]], add=True)
        cp1.wait()
        pltpu.sync_copy(bufs[1], agg_sh.at[dst_v.at[j + 1]], add=True)
        return 0

    lax.fori_loop(0, NBLK // 2, step, 0)
    plsc.subcore_barrier()
    pltpu.sync_copy(agg_sh.at[pl.ds(s * RPT, RPT)],
                    out_hbm.at[c, pl.ds(s * RPT, RPT)])


def _sc_aggregate(g, src_tiles, dst_tiles):
    k = functools.partial(
        pl.kernel,
        out_type=jax.ShapeDtypeStruct((NC, NPAD, H), jnp.float32),
        mesh=_sc_mesh(),
        compiler_params=pltpu.CompilerParams(use_tc_tiling_on_sc=False),
        scratch_types=[
            pltpu.VMEM((NBLK + NBUF, BLK), jnp.int32),
            pltpu.VMEM((NBLK + NBUF, BLK), jnp.int32),
            pltpu.VMEM((BLK, H), jnp.float32),
            pltpu.VMEM((BLK, H), jnp.float32),
            pltpu.VMEM((BLK, H), jnp.float32),
            pltpu.VMEM((BLK, H), jnp.float32),
            pltpu.VMEM((ZR, H), jnp.float32),
            pltpu.VMEM_SHARED((NPAD, H), jnp.float32),
            pltpu.VMEM_SHARED((NPAD, H), jnp.float32),
            pltpu.SemaphoreType.DMA,
            pltpu.SemaphoreType.DMA,
            pltpu.SemaphoreType.DMA,
            pltpu.SemaphoreType.DMA,
        ],
    )(_agg_body)
    return k(g, src_tiles, dst_tiles)


def _tc1_body(x_ref, w_ref, d_ref, g_ref, dis_ref):
    p = jnp.dot(x_ref[...], w_ref[...], preferred_element_type=jnp.float32)
    deg = d_ref[0, :] + d_ref[1, :] + 1.0
    dis = lax.rsqrt(deg)
    g_ref[...] = p * dis[:, None]
    dis_ref[...] = dis


def _tc1(xp, W1, degp):
    return pl.pallas_call(
        _tc1_body,
        grid=(NPAD // BR,),
        in_specs=[
            pl.BlockSpec((BR, F_IN), lambda i: (i, 0)),
            pl.BlockSpec((F_IN, H), lambda i: (0, 0)),
            pl.BlockSpec((NC, BR), lambda i: (0, i)),
        ],
        out_specs=[
            pl.BlockSpec((BR, H), lambda i: (i, 0)),
            pl.BlockSpec((BR,), lambda i: (i,)),
        ],
        out_shape=[
            jax.ShapeDtypeStruct((NPAD, H), jnp.float32),
            jax.ShapeDtypeStruct((NPAD,), jnp.float32),
        ],
    )(xp, W1, degp)


def _tc2_body(a_ref, g_ref, dis_ref, w_ref, b_ref, o_ref):
    dis = dis_ref[...]
    h = a_ref[0] + a_ref[1] + g_ref[...]
    h = jnp.maximum(h * dis[:, None] + b_ref[...][None, :], 0.0)
    p = jnp.dot(h, w_ref[...], preferred_element_type=jnp.float32)
    o_ref[...] = p * dis[:, None]


def _tc2(agg, g1, dis, W2, b1):
    return pl.pallas_call(
        _tc2_body,
        grid=(NPAD // BR,),
        in_specs=[
            pl.BlockSpec((NC, BR, H), lambda i: (0, i, 0)),
            pl.BlockSpec((BR, H), lambda i: (i, 0)),
            pl.BlockSpec((BR,), lambda i: (i,)),
            pl.BlockSpec((H, H), lambda i: (0, 0)),
            pl.BlockSpec((H,), lambda i: (0,)),
        ],
        out_specs=pl.BlockSpec((BR, H), lambda i: (i, 0)),
        out_shape=jax.ShapeDtypeStruct((NPAD, H), jnp.float32),
    )(agg, g1, dis, W2, b1)


def _tc3_body(a_ref, g_ref, dis_ref, w_ref, b2_ref, bc_ref, o_ref):
    dis = dis_ref[...]
    h = a_ref[0] + a_ref[1] + g_ref[...]
    h = jnp.maximum(h * dis[:, None] + b2_ref[...][None, :], 0.0)
    t = jnp.dot(h, w_ref[...], preferred_element_type=jnp.float32)
    t = t + bc_ref[...][None, :]
    m = jnp.max(t, axis=1, keepdims=True)
    lse = m + jnp.log(jnp.sum(jnp.exp(t - m), axis=1, keepdims=True))
    o_ref[...] = t - lse


def _tc3(agg, g2, dis, Wc, b2, bc):
    return pl.pallas_call(
        _tc3_body,
        grid=(NPAD // BR,),
        in_specs=[
            pl.BlockSpec((NC, BR, H), lambda i: (0, i, 0)),
            pl.BlockSpec((BR, H), lambda i: (i, 0)),
            pl.BlockSpec((BR,), lambda i: (i,)),
            pl.BlockSpec((H, C_OUT), lambda i: (0, 0)),
            pl.BlockSpec((H,), lambda i: (0,)),
            pl.BlockSpec((C_OUT,), lambda i: (0,)),
        ],
        out_specs=pl.BlockSpec((BR, C_OUT), lambda i: (i, 0)),
        out_shape=jax.ShapeDtypeStruct((NPAD, C_OUT), jnp.float32),
    )(agg, g2, dis, Wc, b2, bc)


def kernel(x, edge_index, W1, b1, W2, b2, Wc, bc):
    pad = jnp.full((EPAD - E,), N, dtype=jnp.int32)
    src_flat = jnp.concatenate([edge_index[0], pad])
    dst_flat = jnp.concatenate([edge_index[1], pad])
    dummy = jnp.full((NW, NBUF, BLK), N, dtype=jnp.int32)
    src_tiles = jnp.concatenate(
        [src_flat.reshape(NW, NBLK, BLK), dummy], axis=1)
    dst_tiles = jnp.concatenate(
        [dst_flat.reshape(NW, NBLK, BLK), dummy], axis=1)
    dst_tiles_flat = dst_flat.reshape(NW, EPT)

    xp = jnp.pad(x, ((0, NPAD - N), (0, 0)))

    degp = _sc_degree(dst_tiles_flat)
    g1, dis = _tc1(xp, W1, degp)
    agg1 = _sc_aggregate(g1, src_tiles, dst_tiles)
    g2 = _tc2(agg1, g1, dis, W2, b1)
    agg2 = _sc_aggregate(g2, src_tiles, dst_tiles)
    logp = _tc3(agg2, g2, dis, Wc, b2, bc)
    return logp[:N]


# speedup vs baseline: 3.6330x; 1.0284x over previous
import functools

import jax
import jax.numpy as jnp
from jax import lax
from jax.experimental import pallas as pl
from jax.experimental.pallas import tpu as pltpu
from jax.experimental.pallas import tpu_sc as plsc

NC = 2
NS = 16
NW = NC * NS
LANES = 16

N = 10000
F_IN = 128
H = 64
C_OUT = 8
E = 320000

NPAD = 10240
RPT = NPAD // NS
BLK = 128
EPT = 10240
NBLK = EPT // BLK
EPAD = EPT * NW
NBUF = 4
EPTS = (NBLK + NBUF) * BLK
ZR = 128
BR = 512


def _sc_mesh():
    return plsc.VectorSubcoreMesh(core_axis_name="c", subcore_axis_name="s")


def _degree_body(dst_hbm, out_hbm, dst_v, deg_l, acc16, res, sh):
    c = lax.axis_index("c")
    s = lax.axis_index("s")
    w = c * NS + s
    pltpu.sync_copy(dst_hbm.at[w], dst_v)

    zeros = jnp.zeros((LANES,), jnp.float32)

    def zero(k, _):
        deg_l[pl.ds(k * LANES, LANES)] = zeros
        return 0

    lax.fori_loop(0, NPAD // LANES, zero, 0)

    ones = jnp.ones((LANES,), jnp.float32)

    def acc(k, _):
        idx = dst_v[pl.ds(k * LANES, LANES)]
        plsc.addupdate_scatter(deg_l, [idx], ones)
        return 0

    lax.fori_loop(0, EPTS // LANES, acc, 0)

    pltpu.sync_copy(deg_l, sh.at[s])
    plsc.subcore_barrier()

    cpt = NPAD // NS
    for r in range(NS):
        pltpu.sync_copy(sh.at[r, pl.ds(s * cpt, cpt)], acc16.at[r])

    def red(k, _):
        a = acc16[0, pl.ds(k * LANES, LANES)]
        for r in range(1, NS):
            a = a + acc16[r, pl.ds(k * LANES, LANES)]
        res[pl.ds(k * LANES, LANES)] = a
        return 0

    lax.fori_loop(0, cpt // LANES, red, 0)
    pltpu.sync_copy(res, out_hbm.at[c, pl.ds(s * (NPAD // NS), NPAD // NS)])


def _sc_degree(dst_tiles):
    k = functools.partial(
        pl.kernel,
        out_type=jax.ShapeDtypeStruct((NC, NPAD), jnp.float32),
        mesh=_sc_mesh(),
        compiler_params=pltpu.CompilerParams(needs_layout_passes=False),
        scratch_types=[
            pltpu.VMEM((EPTS,), jnp.int32),
            pltpu.VMEM((NPAD,), jnp.float32),
            pltpu.VMEM((NS, NPAD // NS), jnp.float32),
            pltpu.VMEM((NPAD // NS,), jnp.float32),
            pltpu.VMEM_SHARED((NS, NPAD), jnp.float32),
        ],
    )(_degree_body)
    return k(dst_tiles)


def _agg_body(g_hbm, src_hbm, dst_hbm, out_hbm, src_v, dst_v,
              b0, b1, b2, b3, zbuf, g_sh, agg_sh, s0, s1, s2, s3):
    bufs = (b0, b1, b2, b3)
    sems = (s0, s1, s2, s3)
    c = lax.axis_index("c")
    s = lax.axis_index("s")
    w = c * NS + s
    pltpu.sync_copy(src_hbm.at[w], src_v)
    pltpu.sync_copy(dst_hbm.at[w], dst_v)

    zeros = jnp.zeros((LANES,), jnp.float32)

    def zero(r, _):
        for l in range(H // LANES):
            zbuf[r, pl.ds(l * LANES, LANES)] = zeros
        return 0

    lax.fori_loop(0, ZR, zero, 0)
    for q in range(RPT // ZR):
        pltpu.sync_copy(zbuf, agg_sh.at[pl.ds(s * RPT + q * ZR, ZR)])
    plsc.subcore_barrier()

    pltpu.sync_copy(g_hbm.at[pl.ds(s * RPT, RPT)],
                    g_sh.at[pl.ds(s * RPT, RPT)])
    plsc.subcore_barrier()

    def step(j2, _):
        j = 2 * j2
        cp0 = pltpu.async_copy(g_sh.at[src_v.at[j]], bufs[0], sems[0])
        cp1 = pltpu.async_copy(g_sh.at[src_v.at[j + 1]], bufs[1], sems[1])
        cp0.wait()
        pltpu.sync_copy(bufs[0], agg_sh.at[dst_v.at[j]], add=True)
        cp1.wait()
        pltpu.sync_copy(bufs[1], agg_sh.at[dst_v.at[j + 1]], add=True)
        return 0

    lax.fori_loop(0, NBLK // 2, step, 0)
    plsc.subcore_barrier()
    pltpu.sync_copy(agg_sh.at[pl.ds(s * RPT, RPT)],
                    out_hbm.at[c, pl.ds(s * RPT, RPT)])


def _sc_aggregate(g, src_tiles, dst_tiles):
    k = functools.partial(
        pl.kernel,
        out_type=jax.ShapeDtypeStruct((NC, NPAD, H), jnp.float32),
        mesh=_sc_mesh(),
        compiler_params=pltpu.CompilerParams(use_tc_tiling_on_sc=False),
        scratch_types=[
            pltpu.VMEM((NBLK + NBUF, BLK), jnp.int32),
            pltpu.VMEM((NBLK + NBUF, BLK), jnp.int32),
            pltpu.VMEM((BLK, H), jnp.float32),
            pltpu.VMEM((BLK, H), jnp.float32),
            pltpu.VMEM((BLK, H), jnp.float32),
            pltpu.VMEM((BLK, H), jnp.float32),
            pltpu.VMEM((ZR, H), jnp.float32),
            pltpu.VMEM_SHARED((NPAD, H), jnp.float32),
            pltpu.VMEM_SHARED((NPAD, H), jnp.float32),
            pltpu.SemaphoreType.DMA,
            pltpu.SemaphoreType.DMA,
            pltpu.SemaphoreType.DMA,
            pltpu.SemaphoreType.DMA,
        ],
    )(_agg_body)
    return k(g, src_tiles, dst_tiles)


def _tc1_body(x_ref, w_ref, d_ref, g_ref, dis_ref):
    p = jnp.dot(x_ref[...], w_ref[...], preferred_element_type=jnp.float32)
    deg = d_ref[0, :] + d_ref[1, :] + 1.0
    dis = lax.rsqrt(deg)
    g_ref[...] = p * dis[:, None]
    dis_ref[...] = dis


def _tc1(xu, W1, degp):
    return pl.pallas_call(
        _tc1_body,
        grid=(NPAD // BR,),
        in_specs=[
            pl.BlockSpec((BR, F_IN), lambda i: (i, 0)),
            pl.BlockSpec((F_IN, H), lambda i: (0, 0)),
            pl.BlockSpec((NC, BR), lambda i: (0, i)),
        ],
        out_specs=[
            pl.BlockSpec((BR, H), lambda i: (i, 0)),
            pl.BlockSpec((BR,), lambda i: (i,)),
        ],
        out_shape=[
            jax.ShapeDtypeStruct((NPAD, H), jnp.float32),
            jax.ShapeDtypeStruct((NPAD,), jnp.float32),
        ],
    )(xu, W1, degp)


def _tc2_body(a_ref, g_ref, dis_ref, w_ref, b_ref, o_ref):
    dis = dis_ref[...]
    h = a_ref[0] + a_ref[1] + g_ref[...]
    h = jnp.maximum(h * dis[:, None] + b_ref[...][None, :], 0.0)
    p = jnp.dot(h, w_ref[...], preferred_element_type=jnp.float32)
    o_ref[...] = p * dis[:, None]


def _tc2(agg, g1, dis, W2, b1):
    return pl.pallas_call(
        _tc2_body,
        grid=(NPAD // BR,),
        in_specs=[
            pl.BlockSpec((NC, BR, H), lambda i: (0, i, 0)),
            pl.BlockSpec((BR, H), lambda i: (i, 0)),
            pl.BlockSpec((BR,), lambda i: (i,)),
            pl.BlockSpec((H, H), lambda i: (0, 0)),
            pl.BlockSpec((H,), lambda i: (0,)),
        ],
        out_specs=pl.BlockSpec((BR, H), lambda i: (i, 0)),
        out_shape=jax.ShapeDtypeStruct((NPAD, H), jnp.float32),
    )(agg, g1, dis, W2, b1)


def _tc3_body(a_ref, g_ref, dis_ref, w_ref, b2_ref, bc_ref, o_ref):
    dis = dis_ref[...]
    h = a_ref[0] + a_ref[1] + g_ref[...]
    h = jnp.maximum(h * dis[:, None] + b2_ref[...][None, :], 0.0)
    t = jnp.dot(h, w_ref[...], preferred_element_type=jnp.float32)
    t = t + bc_ref[...][None, :]
    m = jnp.max(t, axis=1, keepdims=True)
    lse = m + jnp.log(jnp.sum(jnp.exp(t - m), axis=1, keepdims=True))
    o_ref[...] = t - lse


def _tc3(agg, g2, dis, Wc, b2, bc):
    return pl.pallas_call(
        _tc3_body,
        grid=(NPAD // BR,),
        in_specs=[
            pl.BlockSpec((NC, BR, H), lambda i: (0, i, 0)),
            pl.BlockSpec((BR, H), lambda i: (i, 0)),
            pl.BlockSpec((BR,), lambda i: (i,)),
            pl.BlockSpec((H, C_OUT), lambda i: (0, 0)),
            pl.BlockSpec((H,), lambda i: (0,)),
            pl.BlockSpec((C_OUT,), lambda i: (0,)),
        ],
        out_specs=pl.BlockSpec((BR, C_OUT), lambda i: (i, 0)),
        out_shape=jax.ShapeDtypeStruct((NPAD, C_OUT), jnp.float32),
    )(agg, g2, dis, Wc, b2, bc)


def kernel(x, edge_index, W1, b1, W2, b2, Wc, bc):
    ept_s = (NBLK + NBUF) * BLK
    src_tiles = jnp.pad(
        edge_index[0].reshape(NW, E // NW), ((0, 0), (0, ept_s - E // NW)),
        constant_values=N).reshape(NW, NBLK + NBUF, BLK)
    dst_tiles = jnp.pad(
        edge_index[1].reshape(NW, E // NW), ((0, 0), (0, ept_s - E // NW)),
        constant_values=N).reshape(NW, NBLK + NBUF, BLK)
    dst_tiles_flat = dst_tiles.reshape(NW, ept_s)

    degp = _sc_degree(dst_tiles_flat)
    g1, dis = _tc1(x, W1, degp)
    agg1 = _sc_aggregate(g1, src_tiles, dst_tiles)
    g2 = _tc2(agg1, g1, dis, W2, b1)
    agg2 = _sc_aggregate(g2, src_tiles, dst_tiles)
    logp = _tc3(agg2, g2, dis, Wc, b2, bc)
    return logp[:N]
